# baseline (speedup 1.0000x reference)
import jax
import jax.numpy as jnp
from jax import lax
from jax.experimental import pallas as pl
from jax.experimental.pallas import tpu as pltpu
from jax.experimental.pallas import tpu_sc as plsc

NUM_EMOTIONS = 1000
HIDDEN_DIM = 768
MAX_SEQ_LEN = 32
BATCH = 4096

MP = 128
MCHUNK = 32

NC = 2
NS = 16
NW = NC * NS
BPW = BATCH // NW
BPS = BATCH // NC

IDC = 256
K = 16


def _rows_body(cond_hbm, ids_hbm, out_h_hbm, ids_s, sems):
    cid = lax.axis_index("c")
    base = cid * BPS

    def chunk_body(c, _):
        pltpu.sync_copy(ids_hbm.at[pl.ds(base + c * IDC, IDC)], ids_s)

        def copy_row(g, slot, wait_first):
            pos = base + c * IDC + g
            i = ids_s[g]
            if wait_first:
                pltpu.make_async_copy(
                    cond_hbm.at[pl.ds(i, 1)],
                    out_h_hbm.at[pl.ds(pos, 1)],
                    sems.at[slot]).wait()
            pltpu.async_copy(cond_hbm.at[pl.ds(i, 1)],
                             out_h_hbm.at[pl.ds(pos, 1)],
                             sems.at[slot])

        def grp_body(r, _):
            for b in range(K):
                g = r * K + b
                copy_row(g, b, wait_first=True)
            return _

        @pl.when(c == 0)
        def _():
            for b in range(K):
                copy_row(b, b, wait_first=False)

        @pl.when(c > 0)
        def _():
            for b in range(K):
                copy_row(b, b, wait_first=True)

        lax.fori_loop(1, IDC // K, grp_body, None)
        return _

    lax.fori_loop(0, BPS // IDC, chunk_body, None)

    for b in range(K):
        pltpu.make_async_copy(cond_hbm.at[pl.ds(0, 1)],
                              out_h_hbm.at[pl.ds(b, 1)],
                              sems.at[b]).wait()


def _masks_body(masks_hbm, ids_hbm, out_m_hbm, idx1_v, mrows_v, msem):
    wid = lax.axis_index("s") * NC + lax.axis_index("c")
    base = wid * BPW

    pltpu.sync_copy(ids_hbm.at[pl.ds(base, BPW)], idx1_v)

    for j in range(BPW // MCHUNK):
        pltpu.async_copy(
            masks_hbm.at[idx1_v.at[pl.ds(j * MCHUNK, MCHUNK)]],
            mrows_v, msem).wait()
        pltpu.sync_copy(mrows_v, out_m_hbm.at[pl.ds(base + j * MCHUNK,
                                                    MCHUNK)])


@jax.jit
def _launch(cond, masks, ids):
    rows = pl.kernel(
        _rows_body,
        out_type=jax.ShapeDtypeStruct((BATCH, MAX_SEQ_LEN, HIDDEN_DIM),
                                      jnp.float32),
        mesh=plsc.ScalarSubcoreMesh(axis_name="c", num_cores=NC),
        scratch_types=[
            pltpu.SMEM((IDC,), jnp.int32),
            pltpu.SemaphoreType.DMA((K,)),
        ],
    )
    msk = pl.kernel(
        _masks_body,
        out_type=jax.ShapeDtypeStruct((BATCH, MP), jnp.int32),
        mesh=plsc.VectorSubcoreMesh(core_axis_name="c", subcore_axis_name="s"),
        scratch_types=[
            pltpu.VMEM((BPW,), jnp.int32),
            pltpu.VMEM((MCHUNK, MP), jnp.int32),
            pltpu.SemaphoreType.DMA,
        ],
    )
    return rows(cond, ids), msk(masks, ids)


def kernel(conditioning, attention_masks, emotion_ids):
    masks_pad = jnp.pad(attention_masks, ((0, 0), (0, MP - MAX_SEQ_LEN)))
    hidden, masks_out = _launch(conditioning, masks_pad, emotion_ids)
    return (hidden, masks_out[:, :MAX_SEQ_LEN])


# speedup vs baseline: 40.1073x; 40.1073x over previous
import functools

import jax
import jax.numpy as jnp
from jax import lax
from jax.experimental import pallas as pl
from jax.experimental.pallas import tpu as pltpu
from jax.experimental.pallas import tpu_sc as plsc

NUM_EMOTIONS = 1000
HIDDEN_DIM = 768
MAX_SEQ_LEN = 32
BATCH = 4096
D = MAX_SEQ_LEN * HIDDEN_DIM

MP = 128
MCHUNK = 32
NMC = 4

NC = 2
NS = 16
NW = NC * NS
BPW = BATCH // NW
NBUF = 4
DEPTH = 3
ROUNDS = BPW // NBUF


def _body(cond_hbm, masks_hbm, ids_hbm, ids2_hbm, out_h_hbm, out_m_hbm,
          idx1_v, idx_v, mrows_v, buf_v, gsems, wsems, msems):
    wid = lax.axis_index("s") * NC + lax.axis_index("c")
    base = wid * BPW

    pltpu.sync_copy(ids_hbm.at[pl.ds(base, BPW)], idx1_v)
    pltpu.sync_copy(ids2_hbm.at[pl.ds(base, BPW)], idx_v)

    def start_mask_gather(j):
        pltpu.async_copy(
            masks_hbm.at[idx1_v.at[pl.ds(j * MCHUNK, MCHUNK)]],
            mrows_v.at[j % 2], msems.at[j % 2])

    def finish_mask_chunk(j):
        pltpu.make_async_copy(
            masks_hbm.at[idx1_v.at[pl.ds(j * MCHUNK, MCHUNK)]],
            mrows_v.at[j % 2], msems.at[j % 2]).wait()
        pltpu.sync_copy(mrows_v.at[j % 2],
                        out_m_hbm.at[pl.ds(base + j * MCHUNK, MCHUNK)])
        if j + 2 < NMC:
            start_mask_gather(j + 2)

    start_mask_gather(0)
    start_mask_gather(1)

    def start_gather(g, b):
        pltpu.async_copy(cond_hbm.at[idx_v.at[g]], buf_v.at[b],
                         gsems.at[b])

    def wait_gather(g, b):
        pltpu.make_async_copy(cond_hbm.at[idx_v.at[g]],
                              buf_v.at[b], gsems.at[b]).wait()

    def start_write(g, b):
        pltpu.async_copy(buf_v.at[b], out_h_hbm.at[pl.ds(base + g, 1)],
                         wsems.at[b])

    def wait_write(g, b):
        pltpu.make_async_copy(buf_v.at[b], out_h_hbm.at[pl.ds(base + g, 1)],
                              wsems.at[b]).wait()

    for b in range(DEPTH):
        start_gather(b, b)

    def round_body(o, _):
        for b in range(NBUF):
            g = o * NBUF + b
            wait_gather(g, b)
            start_write(g, b)

            @pl.when(g >= 1)
            def _():
                wait_write(g - 1, (b - 1) % NBUF)

            @pl.when(g + DEPTH < BPW)
            def _():
                start_gather(g + DEPTH, (b + DEPTH) % NBUF)

        for jj in range(NMC - 1):
            @pl.when(o == (jj + 1) * (ROUNDS // NMC))
            def _(jj=jj):
                finish_mask_chunk(jj)
        return _

    lax.fori_loop(0, ROUNDS, round_body, None)

    wait_write(BPW - 1, (BPW - 1) % NBUF)

    finish_mask_chunk(NMC - 1)


@jax.jit
def _launch(cond2d, masks, ids):
    mesh = plsc.VectorSubcoreMesh(core_axis_name="c", subcore_axis_name="s")
    f = pl.kernel(
        _body,
        out_type=(
            jax.ShapeDtypeStruct((BATCH, MAX_SEQ_LEN, HIDDEN_DIM),
                                 jnp.float32),
            jax.ShapeDtypeStruct((BATCH, MP), jnp.int32),
        ),
        mesh=mesh,
        scratch_types=[
            pltpu.VMEM((BPW,), jnp.int32),
            pltpu.VMEM((BPW, 1), jnp.int32),
            pltpu.VMEM((2, MCHUNK, MP), jnp.int32),
            pltpu.VMEM((NBUF, 1, MAX_SEQ_LEN, HIDDEN_DIM), jnp.float32),
            pltpu.SemaphoreType.DMA((NBUF,)),
            pltpu.SemaphoreType.DMA((NBUF,)),
            pltpu.SemaphoreType.DMA((2,)),
        ],
    )
    return f(cond2d, masks, ids, jnp.reshape(ids, (BATCH, 1)))


def kernel(conditioning, attention_masks, emotion_ids):
    masks_pad = jnp.pad(attention_masks, ((0, 0), (0, MP - MAX_SEQ_LEN)))
    hidden, masks_out = _launch(conditioning, masks_pad, emotion_ids)
    return (hidden, masks_out[:, :MAX_SEQ_LEN])


# speedup vs baseline: 40.7592x; 1.0163x over previous
import jax
import jax.numpy as jnp
from jax import lax
from jax.experimental import pallas as pl
from jax.experimental.pallas import tpu as pltpu
from jax.experimental.pallas import tpu_sc as plsc

NUM_EMOTIONS = 1000
HIDDEN_DIM = 768
MAX_SEQ_LEN = 32
BATCH = 4096

NC = 2
NS = 16
NW = NC * NS
BPW = BATCH // NW
NBUF = 4
DEPTH = 3
ROUNDS = BPW // NBUF

EPAD = 1024
MB = 512


def _rows_body(cond_hbm, ids2_hbm, out_h_hbm, idx_v, buf_v, gsems, wsems):
    wid = lax.axis_index("s") * NC + lax.axis_index("c")
    base = wid * BPW

    pltpu.sync_copy(ids2_hbm.at[pl.ds(base, BPW)], idx_v)

    def start_gather(g, b):
        pltpu.async_copy(cond_hbm.at[idx_v.at[g]], buf_v.at[b],
                         gsems.at[b])

    def wait_gather(g, b):
        pltpu.make_async_copy(cond_hbm.at[idx_v.at[g]],
                              buf_v.at[b], gsems.at[b]).wait()

    def start_write(g, b):
        pltpu.async_copy(buf_v.at[b], out_h_hbm.at[pl.ds(base + g, 1)],
                         wsems.at[b])

    def wait_write(g, b):
        pltpu.make_async_copy(buf_v.at[b], out_h_hbm.at[pl.ds(base + g, 1)],
                              wsems.at[b]).wait()

    for b in range(DEPTH):
        start_gather(b, b)

    def round_body(o, _):
        for b in range(NBUF):
            g = o * NBUF + b
            wait_gather(g, b)
            start_write(g, b)

            @pl.when(g >= 1)
            def _():
                wait_write(g - 1, (b - 1) % NBUF)

            @pl.when(g + DEPTH < BPW)
            def _():
                start_gather(g + DEPTH, (b + DEPTH) % NBUF)
        return _

    lax.fori_loop(0, ROUNDS, round_body, None)

    wait_write(BPW - 1, (BPW - 1) % NBUF)


def _masks_tc_body(ids_ref, masks_ref, out_ref):
    onehot = (ids_ref[:, :1] ==
              lax.broadcasted_iota(jnp.int32, (MB, EPAD), 1)
              ).astype(jnp.float32)
    prod = jax.lax.dot_general(onehot, masks_ref[...],
                               (((1,), (0,)), ((), ())),
                               preferred_element_type=jnp.float32)
    out_ref[...] = prod.astype(jnp.int32)


@jax.jit
def _launch(cond, masksf, ids):
    rows = pl.kernel(
        _rows_body,
        out_type=jax.ShapeDtypeStruct((BATCH, MAX_SEQ_LEN, HIDDEN_DIM),
                                      jnp.float32),
        mesh=plsc.VectorSubcoreMesh(core_axis_name="c", subcore_axis_name="s"),
        scratch_types=[
            pltpu.VMEM((BPW, 1), jnp.int32),
            pltpu.VMEM((NBUF, 1, MAX_SEQ_LEN, HIDDEN_DIM), jnp.float32),
            pltpu.SemaphoreType.DMA((NBUF,)),
            pltpu.SemaphoreType.DMA((NBUF,)),
        ],
    )
    masks_out = pl.pallas_call(
        _masks_tc_body,
        out_shape=jax.ShapeDtypeStruct((BATCH, MAX_SEQ_LEN), jnp.int32),
        grid=(BATCH // MB,),
        in_specs=[
            pl.BlockSpec((MB, 1), lambda i: (i, 0)),
            pl.BlockSpec((EPAD, MAX_SEQ_LEN), lambda i: (0, 0)),
        ],
        out_specs=pl.BlockSpec((MB, MAX_SEQ_LEN), lambda i: (i, 0)),
    )(jnp.reshape(ids, (BATCH, 1)), masksf)
    return rows(cond, jnp.reshape(ids, (BATCH, 1))), masks_out


def kernel(conditioning, attention_masks, emotion_ids):
    masksf = jnp.pad(attention_masks.astype(jnp.float32),
                     ((0, EPAD - NUM_EMOTIONS), (0, 0)))
    return _launch(conditioning, masksf, emotion_ids)
